# baseline (speedup 1.0000x reference)
import functools

import jax
import jax.numpy as jnp
from jax import lax
from jax.experimental import pallas as pl
from jax.experimental.pallas import tpu as pltpu
from jax.experimental.pallas import tpu_sc as plsc

B = 16384
NF = 26
D = 64
DP = 128
NC = 2
NS = 16
NW = NC * NS
ROWS_W = B // NW
RB = 8
CHUNK = RB * NF
NBLK = ROWS_W // RB
PER_W = ROWS_W * NF
QR = ROWS_W // 4
L = 16

_mesh = plsc.VectorSubcoreMesh(
    core_axis_name="c", subcore_axis_name="s", num_cores=NC, num_subcores=NS
)


@functools.partial(
    pl.kernel,
    out_type=jax.ShapeDtypeStruct((B, NF * D), jnp.float32),
    mesh=_mesh,
    scratch_types=[
        pltpu.VMEM((PER_W,), jnp.int32),
        pltpu.VMEM((2, CHUNK, DP), jnp.float32),
        pltpu.VMEM((2, RB, NF * D), jnp.float32),
        pltpu.SemaphoreType.DMA((2,)),
        pltpu.SemaphoreType.DMA((2,)),
    ],
    compiler_params=pltpu.CompilerParams(use_tc_tiling_on_sc=True),
)
def _gather_kernel(pep_hbm, xf_hbm, out_hbm, idx_v, pair_v, wb_v,
                   sem_in, sem_out):
    wid = lax.axis_index("s") * NC + lax.axis_index("c")
    row0 = wid * ROWS_W

    pltpu.sync_copy(xf_hbm.at[pl.ds(wid * PER_W, PER_W)], idx_v)

    def gather_start(j, b):
        pltpu.async_copy(
            pep_hbm.at[idx_v.at[pl.ds(j * CHUNK, CHUNK)]], pair_v.at[b],
            sem_in.at[b],
        )

    def gather_wait(b):
        pltpu.make_async_copy(
            pep_hbm.at[idx_v.at[pl.ds(0, CHUNK)]], pair_v.at[b], sem_in.at[b]
        ).wait()

    def wb_start(j, b):
        pltpu.async_copy(
            wb_v.at[b], out_hbm.at[pl.ds(row0 + j * RB, RB)], sem_out.at[b]
        )

    def wb_wait(b):
        pltpu.make_async_copy(
            wb_v.at[b], out_hbm.at[pl.ds(row0, RB)], sem_out.at[b]
        ).wait()

    def compact_block(b):
        for r in range(RB):
            for f in range(NF):
                for k in range(0, D, L):
                    wb_v[b, r, pl.ds(f * D + k, L)] = (
                        pair_v[b, r * NF + f, pl.ds(k, L)]
                    )

    gather_start(0, 0)
    gather_start(1, 1)

    def step(j2, carry):
        for b in range(2):
            j = j2 * 2 + b
            gather_wait(b)

            @pl.when(j2 >= 1)
            def _():
                wb_wait(b)

            compact_block(b)
            wb_start(j, b)
            nxt = lax.select(j + 2 < NBLK, j + 2, 0)
            gather_start(nxt, b)
        return carry

    lax.fori_loop(0, NBLK // 2, step, 0)

    for b in range(2):
        gather_wait(b)
        wb_wait(b)


def kernel(x, pe, dev=0):
    xf = x.reshape(B * NF)
    pe_b, xf = lax.optimization_barrier((pe, xf))
    pep = jnp.pad(pe_b, ((0, 0), (0, DP - D)))
    return _gather_kernel(pep, xf)


# speedup vs baseline: 1.1119x; 1.1119x over previous
import functools

import jax
import jax.numpy as jnp
from jax import lax
from jax.experimental import pallas as pl
from jax.experimental.pallas import tpu as pltpu
from jax.experimental.pallas import tpu_sc as plsc

B = 16384
NF = 26
D = 64
DP = 128
NC = 2
NS = 16
NW = NC * NS
ROWS_W = B // NW
RB = 8
CHUNK = RB * NF
NBLK = ROWS_W // RB
PER_W = ROWS_W * NF
QR = ROWS_W // 4
L = 16

_mesh = plsc.VectorSubcoreMesh(
    core_axis_name="c", subcore_axis_name="s", num_cores=NC, num_subcores=NS
)


@functools.partial(
    pl.kernel,
    out_type=jax.ShapeDtypeStruct((B, NF * D), jnp.float32),
    mesh=_mesh,
    scratch_types=[
        pltpu.VMEM((PER_W,), jnp.int32),
        pltpu.VMEM((2, CHUNK, DP), jnp.float32),
        pltpu.VMEM((2, RB, NF * D), jnp.float32),
        pltpu.SemaphoreType.DMA((2,)),
        pltpu.SemaphoreType.DMA((2,)),
    ],
    compiler_params=pltpu.CompilerParams(use_tc_tiling_on_sc=True),
)
def _gather_kernel(pep_hbm, xf_hbm, out_hbm, idx_v, pair_v, wb_v,
                   sem_in, sem_out):
    wid = lax.axis_index("s") * NC + lax.axis_index("c")
    row0 = wid * ROWS_W

    pltpu.sync_copy(xf_hbm.at[pl.ds(wid * PER_W, PER_W)], idx_v)

    def gather_start(j, b):
        pltpu.async_copy(
            pep_hbm.at[idx_v.at[pl.ds(j * CHUNK, CHUNK)]], pair_v.at[b],
            sem_in.at[b],
        )

    def gather_wait(b):
        pltpu.make_async_copy(
            pep_hbm.at[idx_v.at[pl.ds(0, CHUNK)]], pair_v.at[b], sem_in.at[b]
        ).wait()

    def wb_start(j, b):
        pltpu.async_copy(
            wb_v.at[b], out_hbm.at[pl.ds(row0 + j * RB, RB)], sem_out.at[b]
        )

    def wb_wait(b):
        pltpu.make_async_copy(
            wb_v.at[b], out_hbm.at[pl.ds(row0, RB)], sem_out.at[b]
        ).wait()

    def compact_block(b):
        for r in range(RB):
            for f in range(NF):
                for k in range(0, D, L):
                    wb_v[b, r, pl.ds(f * D + k, L)] = (
                        pair_v[b, r * NF + f, pl.ds(k, L)]
                    )

    gather_start(0, 0)
    gather_start(1, 1)

    def step(j2, carry):
        for b in range(2):
            j = j2 * 2 + b
            gather_wait(b)

            @pl.when(j2 >= 1)
            def _():
                wb_wait(b)

            compact_block(b)
            wb_start(j, b)
            nxt = lax.select(j + 2 < NBLK, j + 2, 0)
            gather_start(nxt, b)
        return carry

    lax.fori_loop(0, NBLK // 2, step, 0)

    for b in range(2):
        gather_wait(b)
        wb_wait(b)


def kernel(x, pe, dev=0):
    pep = jnp.pad(pe, ((0, 0), (0, DP - D)))
    return _gather_kernel(pep, x.reshape(B * NF))
